# baseline (speedup 1.0000x reference)
import jax
import jax.numpy as jnp
from jax.experimental import pallas as pl
from jax.experimental.pallas import tpu as pltpu

BS, T, C_IN = 4, 8, 256
K = 17
NODE_DIM = 1024
HID = 256
PRED_PAD = 128

_PREC = None


def _fused_body(x0_ref, x1_ref, x2_ref, x3_ref, A_ref, Wc_ref, bc_ref,
                Wg_ref, bg_ref, Wzrn_ref, bzrn_ref, Uzrn_ref, Wp_ref, bp_ref,
                feat_ref, pred_ref, h_scr):
    x_refs = (x0_ref, x1_ref, x2_ref, x3_ref)
    j = pl.program_id(0)

    @pl.when(j == 0)
    def _init():
        h_scr[...] = jnp.zeros_like(h_scr)

    AW = jnp.dot(A_ref[:], Wc_ref[:], precision=_PREC,
                 preferred_element_type=jnp.float32)
    ab = jnp.sum(A_ref[:] * bc_ref[:], axis=1, keepdims=True)

    hs = [h_scr[b] for b in range(BS)]
    hu = [jnp.dot(hs[b].astype(jnp.bfloat16), Uzrn_ref[:], precision=_PREC,
                  preferred_element_type=jnp.float32) for b in range(BS)]
    g = [jnp.dot(AW, x_refs[b][0], precision=_PREC,
                 preferred_element_type=jnp.float32) + ab for b in range(BS)]
    gw = [jnp.maximum(
        jnp.dot(g[b].astype(jnp.bfloat16), Wg_ref[:], precision=_PREC,
                preferred_element_type=jnp.float32) + bg_ref[:], 0.0)
        for b in range(BS)]
    xp = [jnp.dot(gw[b].astype(jnp.bfloat16), Wzrn_ref[:], precision=_PREC,
                  preferred_element_type=jnp.float32) + bzrn_ref[:]
          for b in range(BS)]
    for b in range(BS):
        z = jax.nn.sigmoid(xp[b][:, :HID] + hu[b][:, :HID])
        r = jax.nn.sigmoid(xp[b][:, HID:2 * HID] + hu[b][:, HID:2 * HID])
        n = jnp.tanh(xp[b][:, 2 * HID:] + r * hu[b][:, 2 * HID:])
        hn = hs[b] + z * (n - hs[b])
        h_scr[b] = hn
        feat_ref[0, b * K:(b + 1) * K] = hn
        pred_ref[0, b * K:(b + 1) * K] = jnp.dot(
            hn, Wp_ref[:], precision=_PREC,
            preferred_element_type=jnp.float32) + bp_ref[:]


def kernel(x, A, Wconv, bconv, Wg, bg, Wz, Uz, bz, Wr, Ur, br,
           Wn, Un, bn, Wp, bp):
    b, t, c, h, w = x.shape
    xf = x.reshape(b * t, c, h * w)
    Wgb = Wg.astype(jnp.bfloat16)

    Wzrn = jnp.concatenate([Wz, Wr, Wn], axis=1).astype(jnp.bfloat16)
    bzrn = jnp.concatenate([bz, br, bn]).reshape(1, 3 * HID)
    Uzrn = jnp.concatenate([Uz, Ur, Un], axis=1).astype(jnp.bfloat16)
    Wp_pad = jnp.zeros((HID, PRED_PAD), jnp.float32).at[:, :2].set(Wp)
    bp_pad = jnp.zeros((1, PRED_PAD), jnp.float32).at[:, :2].set(bp)

    def _xspec(bb):
        return pl.BlockSpec((1, c, h * w), lambda j, bb=bb: (bb * t + j, 0, 0))

    feat_t, pred_t = pl.pallas_call(
        _fused_body,
        grid=(t,),
        in_specs=[
            _xspec(0), _xspec(1), _xspec(2), _xspec(3),
            pl.BlockSpec((K, K), lambda j: (0, 0)),
            pl.BlockSpec((K, c), lambda j: (0, 0)),
            pl.BlockSpec((1, K), lambda j: (0, 0)),
            pl.BlockSpec((h * w, HID), lambda j: (0, 0)),
            pl.BlockSpec((1, HID), lambda j: (0, 0)),
            pl.BlockSpec((HID, 3 * HID), lambda j: (0, 0)),
            pl.BlockSpec((1, 3 * HID), lambda j: (0, 0)),
            pl.BlockSpec((HID, 3 * HID), lambda j: (0, 0)),
            pl.BlockSpec((HID, PRED_PAD), lambda j: (0, 0)),
            pl.BlockSpec((1, PRED_PAD), lambda j: (0, 0)),
        ],
        out_specs=[
            pl.BlockSpec((1, BS * K, HID), lambda j: (j, 0, 0)),
            pl.BlockSpec((1, BS * K, PRED_PAD), lambda j: (j, 0, 0)),
        ],
        out_shape=[
            jax.ShapeDtypeStruct((t, BS * K, HID), jnp.float32),
            jax.ShapeDtypeStruct((t, BS * K, PRED_PAD), jnp.float32),
        ],
        scratch_shapes=[pltpu.VMEM((BS, K, HID), jnp.float32)],
        compiler_params=pltpu.CompilerParams(
            dimension_semantics=("arbitrary",)),
    )(xf, xf, xf, xf, A, Wconv, bconv.reshape(1, K), Wgb, bg.reshape(1, HID),
      Wzrn, bzrn, Uzrn, Wp_pad, bp_pad)

    feat = feat_t.reshape(t, b, K, HID).transpose(1, 0, 2, 3)
    pred = pred_t[..., :2].reshape(t, b, K, 2).transpose(1, 0, 2, 3)
    return pred, feat


# speedup vs baseline: 1.1926x; 1.1926x over previous
import jax
import jax.numpy as jnp
from jax.experimental import pallas as pl
from jax.experimental.pallas import tpu as pltpu

BS, T, C_IN = 4, 8, 256
K = 17
NODE_DIM = 1024
HID = 256
PRED_PAD = 128

_PREC = None


def _fused_body(x0_ref, x1_ref, x2_ref, x3_ref, A_ref, Wc_ref, bc_ref,
                Wg_ref, bg_ref, Wz_ref, Uz_ref, bz_ref, Wr_ref, Ur_ref,
                br_ref, Wn_ref, Un_ref, bn_ref, Wp_ref, bp_ref,
                feat_ref, pred_ref, h_scr):
    x_refs = (x0_ref, x1_ref, x2_ref, x3_ref)
    j = pl.program_id(0)

    @pl.when(j == 0)
    def _init():
        h_scr[...] = jnp.zeros_like(h_scr)

    AW = jnp.dot(A_ref[:], Wc_ref[:], precision=_PREC,
                 preferred_element_type=jnp.float32)
    ab = jnp.sum(A_ref[:] * bc_ref[:], axis=1, keepdims=True)

    def dt(a, w):
        return jnp.dot(a, w, precision=_PREC,
                       preferred_element_type=jnp.float32)

    hs = [h_scr[b] for b in range(BS)]
    huz = [dt(hs[b], Uz_ref[:]) for b in range(BS)]
    hur = [dt(hs[b], Ur_ref[:]) for b in range(BS)]
    hun = [dt(hs[b], Un_ref[:]) for b in range(BS)]
    g = [dt(AW, x_refs[b][0]) + ab for b in range(BS)]
    gw = [jnp.maximum(dt(g[b].astype(jnp.bfloat16), Wg_ref[:]) + bg_ref[:],
                      0.0) for b in range(BS)]
    xpz = [dt(gw[b], Wz_ref[:]) + bz_ref[:] for b in range(BS)]
    xpr = [dt(gw[b], Wr_ref[:]) + br_ref[:] for b in range(BS)]
    xpn = [dt(gw[b], Wn_ref[:]) + bn_ref[:] for b in range(BS)]
    for b in range(BS):
        z = jax.nn.sigmoid(xpz[b] + huz[b])
        r = jax.nn.sigmoid(xpr[b] + hur[b])
        n = jnp.tanh(xpn[b] + r * hun[b])
        hn = hs[b] + z * (n - hs[b])
        h_scr[b] = hn
        feat_ref[0, b * K:(b + 1) * K] = hn
        pred_ref[0, b * K:(b + 1) * K] = dt(hn, Wp_ref[:]) + bp_ref[:]


def kernel(x, A, Wconv, bconv, Wg, bg, Wz, Uz, bz, Wr, Ur, br,
           Wn, Un, bn, Wp, bp):
    b, t, c, h, w = x.shape
    xf = x.reshape(b * t, c, h * w)
    Wgb = Wg.astype(jnp.bfloat16)


    def _xspec(bb):
        return pl.BlockSpec((1, c, h * w), lambda j, bb=bb: (bb * t + j, 0, 0))

    feat_t, pred_t = pl.pallas_call(
        _fused_body,
        grid=(t,),
        in_specs=[
            _xspec(0), _xspec(1), _xspec(2), _xspec(3),
            pl.BlockSpec((K, K), lambda j: (0, 0)),
            pl.BlockSpec((K, c), lambda j: (0, 0)),
            pl.BlockSpec((1, K), lambda j: (0, 0)),
            pl.BlockSpec((h * w, HID), lambda j: (0, 0)),
            pl.BlockSpec((1, HID), lambda j: (0, 0)),
            pl.BlockSpec((HID, HID), lambda j: (0, 0)),
            pl.BlockSpec((HID, HID), lambda j: (0, 0)),
            pl.BlockSpec((1, HID), lambda j: (0, 0)),
            pl.BlockSpec((HID, HID), lambda j: (0, 0)),
            pl.BlockSpec((HID, HID), lambda j: (0, 0)),
            pl.BlockSpec((1, HID), lambda j: (0, 0)),
            pl.BlockSpec((HID, HID), lambda j: (0, 0)),
            pl.BlockSpec((HID, HID), lambda j: (0, 0)),
            pl.BlockSpec((1, HID), lambda j: (0, 0)),
            pl.BlockSpec((HID, 2), lambda j: (0, 0)),
            pl.BlockSpec((1, 2), lambda j: (0, 0)),
        ],
        out_specs=[
            pl.BlockSpec((1, BS * K, HID), lambda j: (j, 0, 0)),
            pl.BlockSpec((1, BS * K, 2), lambda j: (j, 0, 0)),
        ],
        out_shape=[
            jax.ShapeDtypeStruct((t, BS * K, HID), jnp.float32),
            jax.ShapeDtypeStruct((t, BS * K, 2), jnp.float32),
        ],
        scratch_shapes=[pltpu.VMEM((BS, K, HID), jnp.float32)],
        compiler_params=pltpu.CompilerParams(
            dimension_semantics=("arbitrary",)),
    )(xf, xf, xf, xf, A, Wconv, bconv.reshape(1, K), Wgb, bg.reshape(1, HID),
      Wz, Uz, bz.reshape(1, HID), Wr, Ur, br.reshape(1, HID),
      Wn, Un, bn.reshape(1, HID), Wp, bp.reshape(1, 2))

    feat = feat_t.reshape(t, b, K, HID).transpose(1, 0, 2, 3)
    pred = pred_t.reshape(t, b, K, 2).transpose(1, 0, 2, 3)
    return pred, feat


# speedup vs baseline: 1.2459x; 1.0447x over previous
import jax
import jax.numpy as jnp
from jax.experimental import pallas as pl
from jax.experimental.pallas import tpu as pltpu

BS, T, C_IN = 4, 8, 256
K = 17
NODE_DIM = 1024
HID = 256
PRED_PAD = 128

_PREC = None


def _fused_body(x0_ref, x1_ref, x2_ref, x3_ref, A_ref, Wc_ref, bc_ref,
                Wg_ref, bg_ref, Wz_ref, Uz_ref, bz_ref, Wr_ref, Ur_ref,
                br_ref, Wn_ref, Un_ref, bn_ref, Wp_ref, bp_ref,
                feat_ref, pred_ref, h_scr):
    x_refs = (x0_ref, x1_ref, x2_ref, x3_ref)
    j = pl.program_id(0)

    @pl.when(j == 0)
    def _init():
        h_scr[...] = jnp.zeros_like(h_scr)

    AW = jnp.dot(A_ref[:], Wc_ref[:], precision=_PREC,
                 preferred_element_type=jnp.float32)
    ab = jnp.sum(A_ref[:] * bc_ref[:], axis=1, keepdims=True)

    def dt(a, w):
        return jnp.dot(a, w, precision=_PREC,
                       preferred_element_type=jnp.float32)

    hs = [h_scr[b] for b in range(BS)]
    huz = [dt(hs[b], Uz_ref[:]) for b in range(BS)]
    hur = [dt(hs[b], Ur_ref[:]) for b in range(BS)]
    hun = [dt(hs[b], Un_ref[:]) for b in range(BS)]
    g = [dt(AW, x_refs[b][0]) + ab for b in range(BS)]
    gw = [jnp.maximum(dt(g[b], Wg_ref[:]) + bg_ref[:], 0.0)
          for b in range(BS)]
    xpz = [dt(gw[b], Wz_ref[:]) + bz_ref[:] for b in range(BS)]
    xpr = [dt(gw[b], Wr_ref[:]) + br_ref[:] for b in range(BS)]
    xpn = [dt(gw[b], Wn_ref[:]) + bn_ref[:] for b in range(BS)]
    for b in range(BS):
        z = jax.nn.sigmoid(xpz[b] + huz[b])
        r = jax.nn.sigmoid(xpr[b] + hur[b])
        n = jnp.tanh(xpn[b] + r * hun[b])
        hn = hs[b] + z * (n - hs[b])
        h_scr[b] = hn
        feat_ref[0, b * K:(b + 1) * K] = hn
        pred_ref[0, b * K:(b + 1) * K] = dt(hn, Wp_ref[:]) + bp_ref[:]


def kernel(x, A, Wconv, bconv, Wg, bg, Wz, Uz, bz, Wr, Ur, br,
           Wn, Un, bn, Wp, bp):
    b, t, c, h, w = x.shape
    xf = x.reshape(b * t, c, h * w)


    def _xspec(bb):
        return pl.BlockSpec((1, c, h * w), lambda j, bb=bb: (bb * t + j, 0, 0))

    feat_t, pred_t = pl.pallas_call(
        _fused_body,
        grid=(t,),
        in_specs=[
            _xspec(0), _xspec(1), _xspec(2), _xspec(3),
            pl.BlockSpec((K, K), lambda j: (0, 0)),
            pl.BlockSpec((K, c), lambda j: (0, 0)),
            pl.BlockSpec((1, K), lambda j: (0, 0)),
            pl.BlockSpec((h * w, HID), lambda j: (0, 0)),
            pl.BlockSpec((1, HID), lambda j: (0, 0)),
            pl.BlockSpec((HID, HID), lambda j: (0, 0)),
            pl.BlockSpec((HID, HID), lambda j: (0, 0)),
            pl.BlockSpec((1, HID), lambda j: (0, 0)),
            pl.BlockSpec((HID, HID), lambda j: (0, 0)),
            pl.BlockSpec((HID, HID), lambda j: (0, 0)),
            pl.BlockSpec((1, HID), lambda j: (0, 0)),
            pl.BlockSpec((HID, HID), lambda j: (0, 0)),
            pl.BlockSpec((HID, HID), lambda j: (0, 0)),
            pl.BlockSpec((1, HID), lambda j: (0, 0)),
            pl.BlockSpec((HID, 2), lambda j: (0, 0)),
            pl.BlockSpec((1, 2), lambda j: (0, 0)),
        ],
        out_specs=[
            pl.BlockSpec((1, BS * K, HID), lambda j: (j, 0, 0)),
            pl.BlockSpec((1, BS * K, 2), lambda j: (j, 0, 0)),
        ],
        out_shape=[
            jax.ShapeDtypeStruct((t, BS * K, HID), jnp.float32),
            jax.ShapeDtypeStruct((t, BS * K, 2), jnp.float32),
        ],
        scratch_shapes=[pltpu.VMEM((BS, K, HID), jnp.float32)],
        compiler_params=pltpu.CompilerParams(
            dimension_semantics=("arbitrary",)),
    )(xf, xf, xf, xf, A, Wconv, bconv.reshape(1, K), Wg, bg.reshape(1, HID),
      Wz, Uz, bz.reshape(1, HID), Wr, Ur, br.reshape(1, HID),
      Wn, Un, bn.reshape(1, HID), Wp, bp.reshape(1, 2))

    feat = feat_t.reshape(t, b, K, HID).transpose(1, 0, 2, 3)
    pred = pred_t.reshape(t, b, K, 2).transpose(1, 0, 2, 3)
    return pred, feat


# speedup vs baseline: 1.2714x; 1.0205x over previous
import jax
import jax.numpy as jnp
from jax.experimental import pallas as pl
from jax.experimental.pallas import tpu as pltpu

BS, T, C_IN = 4, 8, 256
K = 17
NODE_DIM = 1024
HID = 256
PRED_PAD = 128

_PREC = None


def _fused_body(x0_ref, x1_ref, x2_ref, x3_ref, A_ref, Wc_ref, bc_ref,
                Wg_ref, bg_ref, Wz_ref, Uz_ref, bz_ref, Wr_ref, Ur_ref,
                br_ref, Wn_ref, Un_ref, bn_ref, Wp_ref, bp_ref,
                feat_ref, pred_ref, h_scr):
    x_refs = (x0_ref, x1_ref, x2_ref, x3_ref)
    j = pl.program_id(0)

    @pl.when(j == 0)
    def _init():
        h_scr[...] = jnp.zeros_like(h_scr)

    AW = jnp.dot(A_ref[:], Wc_ref[:], precision=_PREC,
                 preferred_element_type=jnp.float32)
    ab = jnp.sum(A_ref[:] * bc_ref[:], axis=1, keepdims=True)

    def dt(a, w):
        return jnp.dot(a, w, precision=_PREC,
                       preferred_element_type=jnp.float32)

    g = [[dt(AW, x_refs[b][tt]) + ab for b in range(BS)] for tt in range(2)]
    gw = [[jnp.maximum(dt(g[tt][b], Wg_ref[:]) + bg_ref[:], 0.0)
           for b in range(BS)] for tt in range(2)]
    xpz = [[dt(gw[tt][b], Wz_ref[:]) + bz_ref[:] for b in range(BS)]
           for tt in range(2)]
    xpr = [[dt(gw[tt][b], Wr_ref[:]) + br_ref[:] for b in range(BS)]
           for tt in range(2)]
    xpn = [[dt(gw[tt][b], Wn_ref[:]) + bn_ref[:] for b in range(BS)]
           for tt in range(2)]
    hs = [h_scr[b] for b in range(BS)]
    for tt in range(2):
        huz = [dt(hs[b], Uz_ref[:]) for b in range(BS)]
        hur = [dt(hs[b], Ur_ref[:]) for b in range(BS)]
        hun = [dt(hs[b], Un_ref[:]) for b in range(BS)]
        for b in range(BS):
            z = jax.nn.sigmoid(xpz[tt][b] + huz[b])
            r = jax.nn.sigmoid(xpr[tt][b] + hur[b])
            n = jnp.tanh(xpn[tt][b] + r * hun[b])
            hn = hs[b] + z * (n - hs[b])
            hs[b] = hn
            feat_ref[tt, b * K:(b + 1) * K] = hn
            pred_ref[tt, b * K:(b + 1) * K] = dt(hn, Wp_ref[:]) + bp_ref[:]
    for b in range(BS):
        h_scr[b] = hs[b]


def kernel(x, A, Wconv, bconv, Wg, bg, Wz, Uz, bz, Wr, Ur, br,
           Wn, Un, bn, Wp, bp):
    b, t, c, h, w = x.shape
    xf = x.reshape(b * t, c, h * w)


    def _xspec(bb):
        return pl.BlockSpec((2, c, h * w),
                            lambda j, bb=bb: (bb * (t // 2) + j, 0, 0))

    feat_t, pred_t = pl.pallas_call(
        _fused_body,
        grid=(t // 2,),
        in_specs=[
            _xspec(0), _xspec(1), _xspec(2), _xspec(3),
            pl.BlockSpec((K, K), lambda j: (0, 0)),
            pl.BlockSpec((K, c), lambda j: (0, 0)),
            pl.BlockSpec((1, K), lambda j: (0, 0)),
            pl.BlockSpec((h * w, HID), lambda j: (0, 0)),
            pl.BlockSpec((1, HID), lambda j: (0, 0)),
            pl.BlockSpec((HID, HID), lambda j: (0, 0)),
            pl.BlockSpec((HID, HID), lambda j: (0, 0)),
            pl.BlockSpec((1, HID), lambda j: (0, 0)),
            pl.BlockSpec((HID, HID), lambda j: (0, 0)),
            pl.BlockSpec((HID, HID), lambda j: (0, 0)),
            pl.BlockSpec((1, HID), lambda j: (0, 0)),
            pl.BlockSpec((HID, HID), lambda j: (0, 0)),
            pl.BlockSpec((HID, HID), lambda j: (0, 0)),
            pl.BlockSpec((1, HID), lambda j: (0, 0)),
            pl.BlockSpec((HID, 2), lambda j: (0, 0)),
            pl.BlockSpec((1, 2), lambda j: (0, 0)),
        ],
        out_specs=[
            pl.BlockSpec((2, BS * K, HID), lambda j: (j, 0, 0)),
            pl.BlockSpec((2, BS * K, 2), lambda j: (j, 0, 0)),
        ],
        out_shape=[
            jax.ShapeDtypeStruct((t, BS * K, HID), jnp.float32),
            jax.ShapeDtypeStruct((t, BS * K, 2), jnp.float32),
        ],
        scratch_shapes=[pltpu.VMEM((BS, K, HID), jnp.float32)],
        compiler_params=pltpu.CompilerParams(
            dimension_semantics=("arbitrary",)),
    )(xf, xf, xf, xf, A, Wconv, bconv.reshape(1, K), Wg, bg.reshape(1, HID),
      Wz, Uz, bz.reshape(1, HID), Wr, Ur, br.reshape(1, HID),
      Wn, Un, bn.reshape(1, HID), Wp, bp.reshape(1, 2))

    feat = feat_t.reshape(t, b, K, HID).transpose(1, 0, 2, 3)
    pred = pred_t.reshape(t, b, K, 2).transpose(1, 0, 2, 3)
    return pred, feat
